# baseline (speedup 1.0000x reference)
import functools

import jax
import jax.numpy as jnp
from jax import lax
from jax.experimental import pallas as pl
from jax.experimental.pallas import tpu as pltpu
from jax.experimental.pallas import tpu_sc as plsc

LEVELS = 1024
T = 2048
D = 10000
DP = 10048
SIGNAL_MIN = -5.0
SIGNAL_MAX = 5.0
SLICES = [(0, 3), (3, 9), (9, 12), (12, 15), (15, 18), (18, 21), (21, 24),
          (24, 27), (27, 30)]

NC = 2
NS = 16
NW = NC * NS
TPW = T // NW
L8 = 64
CH = DP // L8
ZCH = DP // 32


def _sc_accum_body(idx_hbm, embed_hbm, keys_hbm, time_hbm, lane_hbm,
                   out_hbm, idx_v, e3, tw, kb, acc, acc2, lane_v, idx4,
                   sem0, sem1):
    wid = lax.axis_index("s") * NC + lax.axis_index("c")
    base = wid * TPW
    pltpu.sync_copy(idx_hbm.at[pl.ds(base, TPW)], idx_v)
    pltpu.sync_copy(lane_hbm, lane_v)

    lv = lane_v[0, :]
    le, lo = plsc.unpack(lv, format=plsc.PackFormat.INTERLEAVED,
                         preferred_element_type=jnp.int16)
    lee, leo = plsc.unpack(le, format=plsc.PackFormat.INTERLEAVED,
                           preferred_element_type=jnp.int32)
    loe, loo = plsc.unpack(lo, format=plsc.PackFormat.INTERLEAVED,
                           preferred_element_type=jnp.int32)
    idx4[0, :] = lee
    idx4[1, :] = leo
    idx4[2, :] = loe
    idx4[3, :] = loo

    def zbody(j, carry):
        acc[pl.ds(j * 32, 32)] = jnp.zeros((32,), jnp.int16)
        return carry

    lax.fori_loop(0, ZCH, zbody, 0)

    slots = [(e3.at[0], tw.at[0], kb.at[0], sem0),
             (e3.at[1], tw.at[1], kb.at[1], sem1)]

    def copies(t, b):
        e3s, tws, kbs, sem = slots[b]
        v = idx_v[t, :]
        return [
            pltpu.make_async_copy(embed_hbm.at[pl.ds(v[0] * DP, DP)],
                                  e3s.at[pl.ds(0, DP)], sem),
            pltpu.make_async_copy(embed_hbm.at[pl.ds(v[1] * DP, DP)],
                                  e3s.at[pl.ds(DP, DP)], sem),
            pltpu.make_async_copy(embed_hbm.at[pl.ds(v[2] * DP, DP)],
                                  e3s.at[pl.ds(2 * DP, DP)], sem),
            pltpu.make_async_copy(time_hbm.at[pl.ds(v[3] * DP, DP)],
                                  tws, sem),
            pltpu.make_async_copy(keys_hbm.at[pl.ds(v[4] * DP, DP)],
                                  kbs, sem),
        ]

    def issue(t, b):
        for c in copies(t, b):
            c.start()

    def wait(t, b):
        for c in copies(t, b):
            c.wait()

    def accum(b):
        e3s, tws, kbs, _ = slots[b]

        def cbody(j, c2):
            s = pl.ds(j * L8, L8)
            e = e3s[pl.ds(j * L8, L8)] + e3s[pl.ds(DP + j * L8, L8)] + \
                e3s[pl.ds(2 * DP + j * L8, L8)]
            p = e * (kbs[s] * tws[s])
            pe, po = plsc.unpack(p, format=plsc.PackFormat.INTERLEAVED,
                                 preferred_element_type=jnp.int16)
            acc[pl.ds(j * L8, 32)] += pe
            acc[pl.ds(j * L8 + 32, 32)] += po
            return c2

        lax.fori_loop(0, CH, cbody, 0)

    issue(0, 0)
    issue(1, 1)

    def tbody(i, carry):
        t = 2 * i
        wait(t, 0)
        accum(0)
        issue(t + 2, 0)
        wait(t + 1, 1)
        accum(1)
        issue(t + 3, 1)
        return carry

    lax.fori_loop(0, TPW // 2 - 1, tbody, 0)
    wait(TPW - 2, 0)
    accum(0)
    wait(TPW - 1, 1)
    accum(1)

    i0 = idx4[0, :]
    i1 = idx4[1, :]
    i2 = idx4[2, :]
    i3 = idx4[3, :]

    def wbody(j, carry):
        dbase = j * L8
        he = acc[pl.ds(dbase, 32)]
        ho = acc[pl.ds(dbase + 32, 32)]
        a, bq = plsc.unpack(he, format=plsc.PackFormat.INTERLEAVED,
                            preferred_element_type=jnp.int32)
        c, dq = plsc.unpack(ho, format=plsc.PackFormat.INTERLEAVED,
                            preferred_element_type=jnp.int32)
        plsc.store_scatter(acc2, [dbase + i0], a.astype(jnp.float32))
        plsc.store_scatter(acc2, [dbase + i1], bq.astype(jnp.float32))
        plsc.store_scatter(acc2, [dbase + i2], c.astype(jnp.float32))
        plsc.store_scatter(acc2, [dbase + i3], dq.astype(jnp.float32))
        return carry

    lax.fori_loop(0, CH, wbody, 0)
    pltpu.sync_copy(acc2.at[pl.ds(0, D)], out_hbm.at[pl.ds(wid * D, D)])


@functools.lru_cache(maxsize=1)
def _get_sc_accum():
    mesh = plsc.VectorSubcoreMesh(
        core_axis_name="c", subcore_axis_name="s",
        num_cores=NC, num_subcores=NS)
    return pl.kernel(
        _sc_accum_body,
        out_type=jax.ShapeDtypeStruct((NW * D,), jnp.float32),
        mesh=mesh,
        scratch_types=[
            pltpu.VMEM((TPW, 16), jnp.int32),
            pltpu.VMEM((2, 3 * DP), jnp.int8),
            pltpu.VMEM((2, DP), jnp.int8),
            pltpu.VMEM((2, DP), jnp.int8),
            pltpu.VMEM((DP,), jnp.int16),
            pltpu.VMEM((DP,), jnp.float32),
            pltpu.VMEM((1, L8), jnp.int8),
            pltpu.VMEM((4, 16), jnp.int32),
            pltpu.SemaphoreType.DMA,
            pltpu.SemaphoreType.DMA,
        ],
        compiler_params=pltpu.CompilerParams(
            use_tc_tiling_on_sc=False, needs_layout_passes=False),
    )


def _tc_combine_body(partial_ref, f_ref, out_ref):
    s = jnp.sum(partial_ref[...], axis=0, keepdims=True)
    v = s * f_ref[...]
    out_ref[...] = jnp.where(v > 0, 1.0, -1.0).astype(jnp.float32)


def _level_idx(x, low, high, num):
    xc = jnp.clip(x, low, high)
    return jnp.round((xc - low) / (high - low) * (num - 1)).astype(jnp.int32)


def kernel(input, feat, embed_w, keys_w, time_w, w0, b0, w1, b1, w2, b2, w3,
           b3, w4, b4, w5, b5, w6, b6, w7, b7, w8, b8):
    eidx = _level_idx(input[:, 1:], SIGNAL_MIN, SIGNAL_MAX, LEVELS)
    tidx = _level_idx(input[:, 0], 0.0, float(T), T).reshape(T, 1)
    trow = jnp.arange(T, dtype=jnp.int32).reshape(T, 1)
    idx16 = jnp.concatenate(
        [eidx, tidx, trow, jnp.zeros((T, 11), jnp.int32)], axis=1)

    pad = ((0, 0), (0, DP - D))
    ei8 = jnp.pad(embed_w.astype(jnp.int8), pad).reshape(-1)
    ki8 = jnp.pad(keys_w.astype(jnp.int8), pad).reshape(-1)
    ti8 = jnp.pad(time_w.astype(jnp.int8), pad).reshape(-1)
    lane = jnp.arange(L8, dtype=jnp.int8).reshape(1, L8)

    partial = _get_sc_accum()(idx16, ei8, ki8, ti8, lane).reshape(NW, D)

    ws = [w0, w1, w2, w3, w4, w5, w6, w7, w8]
    bs = [b0, b1, b2, b3, b4, b5, b6, b7, b8]
    fs = []
    for i, (lo, hi) in enumerate(SLICES):
        p = feat[lo:hi] @ ws[i].T
        fs.append(jnp.cos(p + bs[i]) * jnp.sin(p))
    ftot = fs[0] * (fs[1] + fs[8]) * (fs[2] + fs[3] + fs[4]) * (
        fs[5] + fs[6] + fs[7])

    out = pl.pallas_call(
        _tc_combine_body,
        out_shape=jax.ShapeDtypeStruct((1, D), jnp.float32),
        in_specs=[
            pl.BlockSpec((NW, D), lambda: (0, 0)),
            pl.BlockSpec((1, D), lambda: (0, 0)),
        ],
        out_specs=pl.BlockSpec((1, D), lambda: (0, 0)),
    )(partial, ftot.reshape(1, D))
    return out.reshape(D)


# speedup vs baseline: 1.1209x; 1.1209x over previous
import functools

import jax
import jax.numpy as jnp
from jax import lax
from jax.experimental import pallas as pl
from jax.experimental.pallas import tpu as pltpu
from jax.experimental.pallas import tpu_sc as plsc

LEVELS = 1024
T = 2048
D = 10000
SIGNAL_MIN = -5.0
SIGNAL_MAX = 5.0
SLICES = [(0, 3), (3, 9), (9, 12), (12, 15), (15, 18), (18, 21), (21, 24),
          (24, 27), (27, 30)]

NC = 2
NS = 16
NW = NC * NS
TPW = T // NW
LANES = 16
CH = D // LANES


def _sc_accum_body(idx_hbm, embed_hbm, keys_hbm, time_hbm,
                   out_hbm, idx_v, eb, tw, accg, acca, sem0, sem1):
    wid = lax.axis_index("s") * NC + lax.axis_index("c")
    base = wid * TPW
    pltpu.sync_copy(idx_hbm.at[pl.ds(base, TPW)], idx_v)

    def zbody(j, carry):
        z = jnp.zeros((LANES,), jnp.float32)
        s = pl.ds(j * LANES, LANES)
        acca[0, s] = z
        accg[0, s] = z
        tw[0, s] = z
        return carry

    lax.fori_loop(0, CH, zbody, 0)

    slots = [(eb.at[0], sem0), (eb.at[1], sem1)]

    def copies(t, b):
        ebs, sem = slots[b]
        v = idx_v[t, :]
        return [
            pltpu.make_async_copy(embed_hbm.at[pl.ds(v[0], 1)],
                                  ebs.at[pl.ds(0, 1)], sem),
            pltpu.make_async_copy(embed_hbm.at[pl.ds(v[1], 1)],
                                  ebs.at[pl.ds(1, 1)], sem),
            pltpu.make_async_copy(embed_hbm.at[pl.ds(v[2], 1)],
                                  ebs.at[pl.ds(2, 1)], sem),
            pltpu.make_async_copy(keys_hbm.at[pl.ds(v[4], 1)],
                                  ebs.at[pl.ds(3, 1)], sem),
        ]

    def issue(t, b):
        for c in copies(t, b):
            c.start()

    def wait(t, b):
        for c in copies(t, b):
            c.wait()

    def close_group():
        def gbody(j, c2):
            s = pl.ds(j * LANES, LANES)
            acca[0, s] += accg[0, s] * tw[0, s]
            accg[0, s] = jnp.zeros((LANES,), jnp.float32)
            return c2

        lax.fori_loop(0, CH, gbody, 0)

    def process(t, b):
        ebs, _ = slots[b]
        v = idx_v[t, :]

        @pl.when(v[5] == 1)
        def _():
            close_group()
            pltpu.sync_copy(time_hbm.at[pl.ds(v[3], 1)], tw)

        def cbody(j, c2):
            s = pl.ds(j * LANES, LANES)
            e = ebs[0, s] + ebs[1, s] + ebs[2, s]
            accg[0, s] += e * ebs[3, s]
            return c2

        lax.fori_loop(0, CH, cbody, 0)

    issue(0, 0)
    issue(1, 1)

    def tbody(i, carry):
        t = 2 * i
        wait(t, 0)
        process(t, 0)
        issue(t + 2, 0)
        wait(t + 1, 1)
        process(t + 1, 1)
        issue(t + 3, 1)
        return carry

    lax.fori_loop(0, TPW // 2 - 1, tbody, 0)
    wait(TPW - 2, 0)
    process(TPW - 2, 0)
    wait(TPW - 1, 1)
    process(TPW - 1, 1)
    close_group()
    pltpu.sync_copy(acca, out_hbm.at[pl.ds(wid, 1)])


@functools.lru_cache(maxsize=1)
def _get_sc_accum():
    mesh = plsc.VectorSubcoreMesh(
        core_axis_name="c", subcore_axis_name="s",
        num_cores=NC, num_subcores=NS)
    return pl.kernel(
        _sc_accum_body,
        out_type=jax.ShapeDtypeStruct((NW, D), jnp.float32),
        mesh=mesh,
        scratch_types=[
            pltpu.VMEM((TPW, 16), jnp.int32),
            pltpu.VMEM((2, 4, D), jnp.float32),
            pltpu.VMEM((1, D), jnp.float32),
            pltpu.VMEM((1, D), jnp.float32),
            pltpu.VMEM((1, D), jnp.float32),
            pltpu.SemaphoreType.DMA,
            pltpu.SemaphoreType.DMA,
        ],
        compiler_params=pltpu.CompilerParams(use_tc_tiling_on_sc=False),
    )


def _tc_combine_body(partial_ref, f_ref, out_ref):
    s = jnp.sum(partial_ref[...], axis=0, keepdims=True)
    v = s * f_ref[...]
    out_ref[...] = jnp.where(v > 0, 1.0, -1.0).astype(jnp.float32)


def _level_idx(x, low, high, num):
    xc = jnp.clip(x, low, high)
    return jnp.round((xc - low) / (high - low) * (num - 1)).astype(jnp.int32)


def kernel(input, feat, embed_w, keys_w, time_w, w0, b0, w1, b1, w2, b2, w3,
           b3, w4, b4, w5, b5, w6, b6, w7, b7, w8, b8):
    eidx = _level_idx(input[:, 1:], SIGNAL_MIN, SIGNAL_MAX, LEVELS)
    tidx = _level_idx(input[:, 0], 0.0, float(T), T)

    tl = tidx.reshape(NW, TPW)
    order = jnp.argsort(tl, axis=1, stable=True).astype(jnp.int32)
    permg = order + (jnp.arange(NW, dtype=jnp.int32) * TPW)[:, None]
    pf = permg.reshape(-1)
    tsorted = tl[jnp.arange(NW)[:, None], order]
    prev = jnp.concatenate(
        [jnp.full((NW, 1), -1, jnp.int32), tsorted[:, :-1]], axis=1)
    newflag = (tsorted != prev).astype(jnp.int32).reshape(T, 1)
    idx16 = jnp.concatenate(
        [eidx[pf], tsorted.reshape(T, 1), pf.reshape(T, 1), newflag,
         jnp.zeros((T, 10), jnp.int32)], axis=1)

    partial = _get_sc_accum()(idx16, embed_w, keys_w, time_w)

    ws = [w0, w1, w2, w3, w4, w5, w6, w7, w8]
    bs = [b0, b1, b2, b3, b4, b5, b6, b7, b8]
    fs = []
    for i, (lo, hi) in enumerate(SLICES):
        p = feat[lo:hi] @ ws[i].T
        fs.append(jnp.cos(p + bs[i]) * jnp.sin(p))
    ftot = fs[0] * (fs[1] + fs[8]) * (fs[2] + fs[3] + fs[4]) * (
        fs[5] + fs[6] + fs[7])

    out = pl.pallas_call(
        _tc_combine_body,
        out_shape=jax.ShapeDtypeStruct((1, D), jnp.float32),
        in_specs=[
            pl.BlockSpec((NW, D), lambda: (0, 0)),
            pl.BlockSpec((1, D), lambda: (0, 0)),
        ],
        out_specs=pl.BlockSpec((1, D), lambda: (0, 0)),
    )(partial, ftot.reshape(1, D))
    return out.reshape(D)
